# baseline (speedup 1.0000x reference)
import functools

import jax
import jax.numpy as jnp
from jax.experimental import pallas as pl

NUM_EMBEDDINGS = 8192
EMBEDDING_DIM = 32
COMMITMENT_COST = 0.25
N_TOKENS = 32768
BLOCK_T = 256
KPAD = 128
HALF = NUM_EMBEDDINGS // 2


def _vq_kernel(z_ref, e_ref, q_ref, loss_ref, idx_ref):
    i = pl.program_id(0)
    z = z_ref[...]
    e = e_ref[...]

    z_norm = jnp.sum(z * z, axis=1, keepdims=True)
    e_norm = jnp.sum(e * e, axis=1)
    mm = jax.lax.dot_general(
        z, e, (((1,), (1,)), ((), ())),
        preferred_element_type=jnp.float32)
    distances = (z_norm + e_norm) - 2.0 * mm

    col = jax.lax.broadcasted_iota(jnp.int32, (BLOCK_T, HALF), 1)
    d0 = distances[:, :HALF]
    d1 = distances[:, HALF:]
    v0 = jnp.min(d0, axis=1, keepdims=True)
    v1 = jnp.min(d1, axis=1, keepdims=True)
    i0 = jnp.min(jnp.where(d0 == v0, col, HALF), axis=1)
    i1 = jnp.min(jnp.where(d1 == v1, col, HALF), axis=1) + HALF
    v0b = v0[:, 0].astype(jnp.bfloat16).astype(jnp.float32)
    take1 = v1[:, 0] < v0b
    idx = jnp.where(take1, i1, i0)

    colk = jax.lax.broadcasted_iota(jnp.int32, (BLOCK_T, NUM_EMBEDDINGS), 1)
    one_hot = (colk == idx[:, None]).astype(jnp.float32)
    q = jax.lax.dot_general(
        one_hot, e, (((1,), (0,)), ((), ())),
        preferred_element_type=jnp.float32)

    zd = z[:, :EMBEDDING_DIM]
    diff = q[:, :EMBEDDING_DIM] - zd
    q_ref[...] = zd + diff
    idx_ref[...] = idx

    @pl.when(i == 0)
    def _():
        loss_ref[...] = jnp.zeros_like(loss_ref)

    loss_ref[...] += jnp.sum(diff * diff).reshape(1, 1)

    @pl.when(i == pl.num_programs(0) - 1)
    def _():
        scale = (1.0 + COMMITMENT_COST) / (N_TOKENS * EMBEDDING_DIM)
        loss_ref[...] = loss_ref[...] * scale


@functools.partial(jax.jit, static_argnames=("interpret",))
def kernel(inputs, embedding_weight, interpret=False):
    flat = inputs.reshape(-1, EMBEDDING_DIM)
    zp = jnp.pad(flat, ((0, 0), (0, KPAD - EMBEDDING_DIM)))
    ep = jnp.pad(embedding_weight, ((0, 0), (0, KPAD - EMBEDDING_DIM)))
    grid = (N_TOKENS // BLOCK_T,)
    q, loss, idx = pl.pallas_call(
        _vq_kernel,
        grid=grid,
        in_specs=[
            pl.BlockSpec((BLOCK_T, KPAD), lambda i: (i, 0)),
            pl.BlockSpec((NUM_EMBEDDINGS, KPAD), lambda i: (0, 0)),
        ],
        out_specs=[
            pl.BlockSpec((BLOCK_T, EMBEDDING_DIM), lambda i: (i, 0)),
            pl.BlockSpec((1, 1), lambda i: (0, 0)),
            pl.BlockSpec((BLOCK_T,), lambda i: (i,)),
        ],
        out_shape=[
            jax.ShapeDtypeStruct((N_TOKENS, EMBEDDING_DIM), jnp.float32),
            jax.ShapeDtypeStruct((1, 1), jnp.float32),
            jax.ShapeDtypeStruct((N_TOKENS,), jnp.int32),
        ],
        interpret=interpret,
    )(zp, ep)
    return q, loss[0, 0], idx


# speedup vs baseline: 1.0907x; 1.0907x over previous
import functools

import jax
import jax.numpy as jnp
from jax.experimental import pallas as pl

NUM_EMBEDDINGS = 8192
EMBEDDING_DIM = 32
COMMITMENT_COST = 0.25
N_TOKENS = 32768
BLOCK_T = 512
KPAD = 128
HALF = NUM_EMBEDDINGS // 2


def _vq_kernel(z_ref, e_ref, q_ref, loss_ref, idx_ref):
    i = pl.program_id(0)
    z = z_ref[...]
    e = e_ref[...]

    z_norm = jnp.sum(z * z, axis=1, keepdims=True)
    e_norm = jnp.sum(e * e, axis=1)
    mm = jax.lax.dot_general(
        z, e, (((1,), (1,)), ((), ())),
        preferred_element_type=jnp.float32)
    distances = (z_norm + e_norm) - 2.0 * mm

    col = jax.lax.broadcasted_iota(jnp.int32, (BLOCK_T, HALF), 1)
    d0 = distances[:, :HALF]
    d1 = distances[:, HALF:]
    v0 = jnp.min(d0, axis=1, keepdims=True)
    v1 = jnp.min(d1, axis=1, keepdims=True)
    i0 = jnp.min(jnp.where(d0 == v0, col, HALF), axis=1)
    i1 = jnp.min(jnp.where(d1 == v1, col, HALF), axis=1) + HALF
    v0b = v0[:, 0].astype(jnp.bfloat16).astype(jnp.float32)
    take1 = v1[:, 0] < v0b
    idx = jnp.where(take1, i1, i0)

    colk = jax.lax.broadcasted_iota(jnp.int32, (BLOCK_T, NUM_EMBEDDINGS), 1)
    one_hot = (colk == idx[:, None]).astype(jnp.float32)
    q = jax.lax.dot_general(
        one_hot, e, (((1,), (0,)), ((), ())),
        preferred_element_type=jnp.float32)

    zd = z[:, :EMBEDDING_DIM]
    diff = q[:, :EMBEDDING_DIM] - zd
    q_ref[...] = zd + diff
    idx_ref[...] = idx

    @pl.when(i == 0)
    def _():
        loss_ref[...] = jnp.zeros_like(loss_ref)

    loss_ref[...] += jnp.sum(diff * diff).reshape(1, 1)

    @pl.when(i == pl.num_programs(0) - 1)
    def _():
        scale = (1.0 + COMMITMENT_COST) / (N_TOKENS * EMBEDDING_DIM)
        loss_ref[...] = loss_ref[...] * scale


@functools.partial(jax.jit, static_argnames=("interpret",))
def kernel(inputs, embedding_weight, interpret=False):
    flat = inputs.reshape(-1, EMBEDDING_DIM)
    zp = jnp.pad(flat, ((0, 0), (0, KPAD - EMBEDDING_DIM)))
    ep = jnp.pad(embedding_weight, ((0, 0), (0, KPAD - EMBEDDING_DIM)))
    grid = (N_TOKENS // BLOCK_T,)
    q, loss, idx = pl.pallas_call(
        _vq_kernel,
        grid=grid,
        in_specs=[
            pl.BlockSpec((BLOCK_T, KPAD), lambda i: (i, 0)),
            pl.BlockSpec((NUM_EMBEDDINGS, KPAD), lambda i: (0, 0)),
        ],
        out_specs=[
            pl.BlockSpec((BLOCK_T, EMBEDDING_DIM), lambda i: (i, 0)),
            pl.BlockSpec((1, 1), lambda i: (0, 0)),
            pl.BlockSpec((BLOCK_T,), lambda i: (i,)),
        ],
        out_shape=[
            jax.ShapeDtypeStruct((N_TOKENS, EMBEDDING_DIM), jnp.float32),
            jax.ShapeDtypeStruct((1, 1), jnp.float32),
            jax.ShapeDtypeStruct((N_TOKENS,), jnp.int32),
        ],
        interpret=interpret,
    )(zp, ep)
    return q, loss[0, 0], idx


# speedup vs baseline: 1.1116x; 1.0192x over previous
import functools

import jax
import jax.numpy as jnp
from jax.experimental import pallas as pl

NUM_EMBEDDINGS = 8192
EMBEDDING_DIM = 32
COMMITMENT_COST = 0.25
N_TOKENS = 32768
BLOCK_T = 1024
KPAD = 128
HALF = NUM_EMBEDDINGS // 2


def _vq_kernel(z_ref, e_ref, q_ref, loss_ref, idx_ref):
    i = pl.program_id(0)
    z = z_ref[...]
    e = e_ref[...]

    z_norm = jnp.sum(z * z, axis=1, keepdims=True)
    e_norm = jnp.sum(e * e, axis=1)
    mm = jax.lax.dot_general(
        z, e, (((1,), (1,)), ((), ())),
        preferred_element_type=jnp.float32)
    distances = (z_norm + e_norm) - 2.0 * mm

    col = jax.lax.broadcasted_iota(jnp.int32, (BLOCK_T, HALF), 1)
    d0 = distances[:, :HALF]
    d1 = distances[:, HALF:]
    v0 = jnp.min(d0, axis=1, keepdims=True)
    v1 = jnp.min(d1, axis=1, keepdims=True)
    i0 = jnp.min(jnp.where(d0 == v0, col, HALF), axis=1)
    i1 = jnp.min(jnp.where(d1 == v1, col, HALF), axis=1) + HALF
    v0b = v0[:, 0].astype(jnp.bfloat16).astype(jnp.float32)
    take1 = v1[:, 0] < v0b
    idx = jnp.where(take1, i1, i0)

    colk = jax.lax.broadcasted_iota(jnp.int32, (BLOCK_T, NUM_EMBEDDINGS), 1)
    one_hot = (colk == idx[:, None]).astype(jnp.float32)
    q = jax.lax.dot_general(
        one_hot, e, (((1,), (0,)), ((), ())),
        preferred_element_type=jnp.float32)

    zd = z[:, :EMBEDDING_DIM]
    diff = q[:, :EMBEDDING_DIM] - zd
    q_ref[...] = zd + diff
    idx_ref[...] = idx

    @pl.when(i == 0)
    def _():
        loss_ref[...] = jnp.zeros_like(loss_ref)

    loss_ref[...] += jnp.sum(diff * diff).reshape(1, 1)

    @pl.when(i == pl.num_programs(0) - 1)
    def _():
        scale = (1.0 + COMMITMENT_COST) / (N_TOKENS * EMBEDDING_DIM)
        loss_ref[...] = loss_ref[...] * scale


@functools.partial(jax.jit, static_argnames=("interpret",))
def kernel(inputs, embedding_weight, interpret=False):
    flat = inputs.reshape(-1, EMBEDDING_DIM)
    zp = jnp.pad(flat, ((0, 0), (0, KPAD - EMBEDDING_DIM)))
    ep = jnp.pad(embedding_weight, ((0, 0), (0, KPAD - EMBEDDING_DIM)))
    grid = (N_TOKENS // BLOCK_T,)
    q, loss, idx = pl.pallas_call(
        _vq_kernel,
        grid=grid,
        in_specs=[
            pl.BlockSpec((BLOCK_T, KPAD), lambda i: (i, 0)),
            pl.BlockSpec((NUM_EMBEDDINGS, KPAD), lambda i: (0, 0)),
        ],
        out_specs=[
            pl.BlockSpec((BLOCK_T, EMBEDDING_DIM), lambda i: (i, 0)),
            pl.BlockSpec((1, 1), lambda i: (0, 0)),
            pl.BlockSpec((BLOCK_T,), lambda i: (i,)),
        ],
        out_shape=[
            jax.ShapeDtypeStruct((N_TOKENS, EMBEDDING_DIM), jnp.float32),
            jax.ShapeDtypeStruct((1, 1), jnp.float32),
            jax.ShapeDtypeStruct((N_TOKENS,), jnp.int32),
        ],
        interpret=interpret,
    )(zp, ep)
    return q, loss[0, 0], idx
